# speedup vs baseline: 3.4612x; 3.4612x over previous
import functools

import jax
import jax.numpy as jnp
from jax import lax
from jax.experimental import pallas as pl
from jax.experimental.pallas import tpu as pltpu
from jax.experimental.pallas import tpu_sc as plsc

VOCAB = 100000
DIM = 128

NC = 2
NS = 16
NW = NC * NS

CHUNK = 128
NBUF = 4


def _make_kernel(total_rows: int):
    rows_per_w = total_rows // NW
    cpw = rows_per_w // CHUNK

    mesh = plsc.VectorSubcoreMesh(core_axis_name="c", subcore_axis_name="s")

    scratch = [pltpu.VMEM((cpw, CHUNK), jnp.int32)]
    scratch += [pltpu.VMEM((CHUNK, DIM), jnp.float32) for _ in range(NBUF)]
    scratch += [pltpu.SemaphoreType.DMA for _ in range(NBUF)]

    @functools.partial(
        pl.kernel,
        out_type=jax.ShapeDtypeStruct((total_rows // CHUNK, CHUNK, DIM),
                                      jnp.float32),
        mesh=mesh,
        scratch_types=scratch,
    )
    def emb_kernel(x_hbm, tab_hbm, out_hbm, idx_v, *rest):
        bufs = rest[:NBUF]
        sems = rest[NBUF:2 * NBUF]

        wid = lax.axis_index("s") * NC + lax.axis_index("c")
        pltpu.sync_copy(x_hbm.at[wid], idx_v)

        chunk0 = wid * cpw

        def start_gather(b, g):
            pltpu.async_copy(tab_hbm.at[idx_v.at[g]], bufs[b], sems[b])

        def wait_gather(b):
            pltpu.make_async_copy(tab_hbm.at[idx_v.at[0]], bufs[b],
                                  sems[b]).wait()

        for b in range(NBUF):
            start_gather(b, b)

        @pl.loop(0, cpw - NBUF, step=NBUF)
        def _(o):
            for b in range(NBUF):
                g = o + b
                wait_gather(b)
                pltpu.sync_copy(bufs[b], out_hbm.at[chunk0 + g])
                start_gather(b, g + NBUF)

        for b in range(NBUF):
            g = cpw - NBUF + b
            wait_gather(b)
            pltpu.sync_copy(bufs[b], out_hbm.at[chunk0 + g])

    return emb_kernel


def kernel(x, emb):
    n, m = x.shape
    total = n * m
    idx = x.reshape(NW, total // (NW * CHUNK), CHUNK).astype(jnp.int32)
    out = _make_kernel(total)(idx, emb)
    return out.reshape(n, m, DIM)
